# baseline (speedup 1.0000x reference)
import functools

import jax
import jax.numpy as jnp
from jax import lax
from jax.experimental import pallas as pl
from jax.experimental.pallas import tpu as pltpu
from jax.experimental.pallas import tpu_sc as plsc

_FIXEDLEN = 4096
_PAD_VALUE = 1.0
_B = 16
_HALF = _FIXEDLEN // 2
_LANES = 16
_WIN = _FIXEDLEN + _LANES
_BUF = 2 * _FIXEDLEN + _WIN + 2 * _LANES


@functools.lru_cache(maxsize=None)
def _make_densify(total: int):
    mesh = plsc.VectorSubcoreMesh(
        core_axis_name="c", subcore_axis_name="s", num_cores=1)

    @functools.partial(
        pl.kernel,
        mesh=mesh,
        out_type=jax.ShapeDtypeStruct((_B, _FIXEDLEN), jnp.float32),
        scratch_types=[
            pltpu.VMEM((32,), jnp.int32),
            pltpu.VMEM((_BUF,), jnp.float32),
            pltpu.VMEM((_FIXEDLEN,), jnp.float32),
            pltpu.SemaphoreType.DMA,
        ],
    )
    def densify(flat_hbm, cu_hbm, out_hbm, cu_v, buf_v, out_v, sem):
        row = lax.axis_index("s")

        pltpu.sync_copy(cu_hbm, cu_v.at[pl.ds(0, _B + 1)])
        cuw = cu_v[pl.ds(row, _LANES)]
        start = cuw[0]
        lim = cuw[1] - start

        aligned = jnp.minimum((start // 8) * 8, total - _WIN)
        win = pl.multiple_of(aligned, 8)
        r = start - win
        pltpu.sync_copy(flat_hbm.at[pl.ds(win, _WIN)], buf_v.at[pl.ds(0, _WIN)])

        def half(lo):
            @plsc.parallel_loop(lo, lo + _HALF, _LANES, unroll=2)
            def body(off):
                vals = buf_v[pl.ds(r + off, _LANES)]
                pos = lax.iota(jnp.int32, _LANES) + off
                out_v[pl.ds(off, _LANES)] = jnp.where(
                    pos < lim, vals, jnp.float32(_PAD_VALUE))

        half(0)
        h0 = pltpu.make_async_copy(
            out_v.at[pl.ds(0, _HALF)],
            out_hbm.at[row, pl.ds(0, _HALF)], sem)
        h0.start()
        half(_HALF)
        h1 = pltpu.make_async_copy(
            out_v.at[pl.ds(_HALF, _HALF)],
            out_hbm.at[row, pl.ds(_HALF, _HALF)], sem)
        h1.start()
        h0.wait()
        h1.wait()

    return densify


def kernel(flat_values, cu_seqlens):
    return _make_densify(flat_values.shape[0])(
        flat_values, cu_seqlens.astype(jnp.int32))


# speedup vs baseline: 1.0154x; 1.0154x over previous
import functools

import jax
import jax.numpy as jnp
from jax import lax
from jax.experimental import pallas as pl
from jax.experimental.pallas import tpu as pltpu
from jax.experimental.pallas import tpu_sc as plsc

_FIXEDLEN = 4096
_PAD_VALUE = 1.0
_B = 16
_HALF = _FIXEDLEN // 2
_LANES = 16
_WIN = _FIXEDLEN + _LANES
_BUF = 2 * _FIXEDLEN + _WIN + 2 * _LANES


@functools.lru_cache(maxsize=None)
def _make_densify(total: int):
    mesh = plsc.VectorSubcoreMesh(
        core_axis_name="c", subcore_axis_name="s", num_cores=1)

    @functools.partial(
        pl.kernel,
        mesh=mesh,
        out_type=jax.ShapeDtypeStruct((_B, _FIXEDLEN), jnp.float32),
        scratch_types=[
            pltpu.VMEM((32,), jnp.int32),
            pltpu.VMEM((_BUF,), jnp.float32),
            pltpu.VMEM((_FIXEDLEN,), jnp.float32),
            pltpu.SemaphoreType.DMA,
        ],
    )
    def densify(flat_hbm, cu_hbm, out_hbm, cu_v, buf_v, out_v, sem):
        row = lax.axis_index("s")

        pltpu.sync_copy(cu_hbm, cu_v.at[pl.ds(0, _B + 1)])
        cuw = cu_v[pl.ds(row, _LANES)]
        start = cuw[0]
        lim = cuw[1] - start

        aligned = jnp.minimum((start // 8) * 8, total - _WIN)
        win = pl.multiple_of(aligned, 8)
        r = start - win
        pltpu.sync_copy(flat_hbm.at[pl.ds(win, _WIN)], buf_v.at[pl.ds(0, _WIN)])

        def half(lo):
            @plsc.parallel_loop(lo, lo + _HALF, _LANES, unroll=8)
            def body(off):
                vals = buf_v[pl.ds(r + off, _LANES)]
                pos = lax.iota(jnp.int32, _LANES) + off
                out_v[pl.ds(off, _LANES)] = jnp.where(
                    pos < lim, vals, jnp.float32(_PAD_VALUE))

        half(0)
        h0 = pltpu.make_async_copy(
            out_v.at[pl.ds(0, _HALF)],
            out_hbm.at[row, pl.ds(0, _HALF)], sem)
        h0.start()
        half(_HALF)
        h1 = pltpu.make_async_copy(
            out_v.at[pl.ds(_HALF, _HALF)],
            out_hbm.at[row, pl.ds(_HALF, _HALF)], sem)
        h1.start()
        h0.wait()
        h1.wait()

    return densify


def kernel(flat_values, cu_seqlens):
    return _make_densify(flat_values.shape[0])(
        flat_values, cu_seqlens.astype(jnp.int32))
